# baseline (speedup 1.0000x reference)
import functools

import jax
import jax.numpy as jnp
from jax import lax
from jax.experimental import pallas as pl
from jax.experimental.pallas import tpu as pltpu
from jax.experimental.pallas import tpu_sc as plsc

E = 8
D = 768
F = 3072
K = 2
T = 2048
N = T * K
B = 128
NB = N // B + E
NP = NB * B
NEG = -1e30
NW = 32
TPW = T // NW
L = 16
CH = 32
NCH = TPW // CH
GS = 128
G = N // GS

FB = float(B)


def _gate_route_kernel(x_ref, wg_ref, bg_ref, s0_ref, destT_ref, meta_ref):
    x = x_ref[...]
    logits = jnp.dot(x, wg_ref[...], preferred_element_type=jnp.float32)
    logits = logits + bg_ref[...]
    l8i = (lax.broadcasted_iota(jnp.int32, (E, E), 0)
           <= lax.broadcasted_iota(jnp.int32, (E, E), 1)).astype(jnp.float32)
    m1 = jnp.max(logits, axis=1, keepdims=True)
    eq1 = (logits == m1).astype(jnp.float32)
    cs1 = jnp.dot(eq1, l8i, preferred_element_type=jnp.float32)
    oh_a = jnp.where(cs1 == 1.0, eq1, 0.0)
    l2 = logits + NEG * oh_a
    m2 = jnp.max(l2, axis=1, keepdims=True)
    eq2 = (l2 == m2).astype(jnp.float32)
    cs2 = jnp.dot(eq2, l8i, preferred_element_type=jnp.float32)
    oh_b = jnp.where(cs2 == 1.0, eq2, 0.0)
    s0 = 1.0 / (1.0 + jnp.exp(m2 - m1))
    s0_ref[...] = jnp.broadcast_to(s0, (T, L))
    oh = jnp.concatenate([oh_a, oh_b], axis=0)

    tril = (lax.broadcasted_iota(jnp.int32, (GS, GS), 0)
            >= lax.broadcasted_iota(jnp.int32, (GS, GS), 1)).astype(jnp.float32)
    segs = []
    csums = []
    totals = []
    for g in range(G):
        seg = oh[g * GS:(g + 1) * GS]
        cs = jnp.dot(tril, seg, preferred_element_type=jnp.float32)
        segs.append(seg)
        csums.append(cs)
        totals.append(cs[GS - 1:GS, :])
    tot = jnp.concatenate(totals, axis=0)
    trilg = (lax.broadcasted_iota(jnp.int32, (G, G), 0)
             > lax.broadcasted_iota(jnp.int32, (G, G), 1)).astype(jnp.float32)
    goff = jnp.dot(trilg, tot, preferred_element_type=jnp.float32)
    counts = jnp.sum(tot, axis=0, keepdims=True)
    nblk = jnp.floor((counts + (FB - 1.0)) / FB)
    padded = nblk * FB
    u8 = (lax.broadcasted_iota(jnp.int32, (E, E), 0)
          < lax.broadcasted_iota(jnp.int32, (E, E), 1)).astype(jnp.float32)
    poff = jnp.dot(padded, u8, preferred_element_type=jnp.float32)

    dparts = []
    for g in range(G):
        rank = csums[g] - segs[g] + goff[g:g + 1, :]
        dval = jnp.sum(segs[g] * (rank + poff), axis=1, keepdims=True)
        dparts.append(dval)
    dest = jnp.concatenate(dparts, axis=0)
    dm = jnp.concatenate(
        [dest[:T], dest[T:], jnp.zeros((T, 126), jnp.float32)], axis=1)
    dt = jnp.swapaxes(dm, 0, 1)
    destT_ref[...] = dt[:K, :].astype(jnp.int32)

    l8 = (lax.broadcasted_iota(jnp.int32, (E, E), 0)
          <= lax.broadcasted_iota(jnp.int32, (E, E), 1)).astype(jnp.float32)
    pbe = jnp.dot(nblk, l8, preferred_element_type=jnp.float32)
    nba = jnp.sum(nblk, axis=1, keepdims=True)
    jcol = lax.broadcasted_iota(jnp.int32, (NB, 1), 0).astype(jnp.float32)
    be_raw = jnp.sum((pbe <= jcol).astype(jnp.float32), axis=1, keepdims=True)
    be = jnp.minimum(be_raw, float(E - 1))
    last_e = jnp.sum(jnp.where(jcol == nba - 1.0, be, 0.0), axis=0, keepdims=True)
    bev = jnp.where(jcol < nba, be, last_e)
    mm = jnp.concatenate([bev, nba], axis=0)
    meta_ref[...] = jnp.broadcast_to(mm, (NB + 1, 128)).astype(jnp.int32)


def _gate_route(moe_inp, Wg, bg):
    return pl.pallas_call(
        _gate_route_kernel,
        grid=(1,),
        in_specs=[
            pl.BlockSpec((T, D), lambda i: (0, 0)),
            pl.BlockSpec((D, E), lambda i: (0, 0)),
            pl.BlockSpec((1, E), lambda i: (0, 0)),
        ],
        out_specs=[
            pl.BlockSpec((T, L), lambda i: (0, 0)),
            pl.BlockSpec((K, T), lambda i: (0, 0)),
            pl.BlockSpec((NB + 1, 128), lambda i: (0, 0)),
        ],
        out_shape=[
            jax.ShapeDtypeStruct((T, L), jnp.float32),
            jax.ShapeDtypeStruct((K, T), jnp.int32),
            jax.ShapeDtypeStruct((NB + 1, 128), jnp.int32),
        ],
    )(moe_inp, Wg, bg.reshape(1, E))


@functools.cache
def _build_dispatch():
    mesh = plsc.VectorSubcoreMesh(core_axis_name="c", subcore_axis_name="s")
    return pl.kernel(
        _dispatch_body,
        out_type=jax.ShapeDtypeStruct((NP, D), jnp.float32),
        mesh=mesh,
        scratch_types=[
            pltpu.VMEM((TPW, D), jnp.float32),
            pltpu.VMEM((NCH, CH), jnp.int32),
            pltpu.VMEM((NCH, CH), jnp.int32),
            pltpu.SemaphoreType.DMA,
            pltpu.SemaphoreType.DMA,
            pltpu.SemaphoreType.DMA,
            pltpu.SemaphoreType.DMA,
        ],
    )


def _dispatch_body(x_hbm, destT_hbm, xs_hbm, rows_v, ia_v, ib_v,
                   sem_x, sem_i, sem_a, sem_b):
    wid = lax.axis_index("c") * 16 + lax.axis_index("s")
    base = wid * TPW
    cps_x = [
        pltpu.async_copy(
            x_hbm.at[pl.ds(base + c * CH, CH)],
            rows_v.at[pl.ds(c * CH, CH)], sem_x)
        for c in range(NCH)
    ]
    cps_i = [
        pltpu.async_copy(
            destT_hbm.at[k, pl.ds(base + c * CH, CH)],
            (ia_v, ib_v)[k].at[c], sem_i)
        for k in range(K) for c in range(NCH)
    ]
    for cp in cps_i:
        cp.wait()
    outs = []
    for c in range(NCH):
        cps_x[c].wait()
        sl = pl.ds(c * CH, CH)
        outs.append(pltpu.async_copy(
            rows_v.at[sl], xs_hbm.at[ia_v.at[c]], sem_a))
        outs.append(pltpu.async_copy(
            rows_v.at[sl], xs_hbm.at[ib_v.at[c]], sem_b))
    for cp in outs:
        cp.wait()


def _ffn_kernel(meta_ref, x_ref, w1_ref, b1_ref, w2_ref, b2_ref, y_ref):
    j = pl.program_id(0)

    @pl.when(j < meta_ref[NB, 0])
    def _():
        x = x_ref[...]
        h = jnp.dot(x, w1_ref[0], preferred_element_type=jnp.float32)
        h = jnp.maximum(h + b1_ref[0], 0.0)
        y = jnp.dot(h, w2_ref[0], preferred_element_type=jnp.float32)
        y_ref[...] = y + b2_ref[0]


def _grouped_ffn(meta, xs, W1, b1, W2, b2):
    grid_spec = pltpu.PrefetchScalarGridSpec(
        num_scalar_prefetch=1,
        grid=(NB,),
        in_specs=[
            pl.BlockSpec((B, D), lambda j, m: (jnp.minimum(j, m[NB, 0] - 1), 0)),
            pl.BlockSpec((1, D, F), lambda j, m: (m[j, 0], 0, 0)),
            pl.BlockSpec((1, 1, F), lambda j, m: (m[j, 0], 0, 0)),
            pl.BlockSpec((1, F, D), lambda j, m: (m[j, 0], 0, 0)),
            pl.BlockSpec((1, 1, D), lambda j, m: (m[j, 0], 0, 0)),
        ],
        out_specs=pl.BlockSpec((B, D), lambda j, m: (jnp.minimum(j, m[NB, 0] - 1), 0)),
    )
    return pl.pallas_call(
        _ffn_kernel,
        grid_spec=grid_spec,
        out_shape=jax.ShapeDtypeStruct((NP, D), jnp.float32),
    )(meta, xs, W1, b1.reshape(E, 1, F), W2, b2.reshape(E, 1, D))


@functools.cache
def _build_combine():
    mesh = plsc.VectorSubcoreMesh(core_axis_name="c", subcore_axis_name="s")
    return pl.kernel(
        _combine_body,
        out_type=jax.ShapeDtypeStruct((T, D), jnp.float32),
        mesh=mesh,
        scratch_types=[
            pltpu.VMEM((TPW, D), jnp.float32),
            pltpu.VMEM((TPW, D), jnp.float32),
            pltpu.VMEM((NCH, CH), jnp.int32),
            pltpu.VMEM((NCH, CH), jnp.int32),
            pltpu.VMEM((TPW, L), jnp.float32),
            pltpu.SemaphoreType.DMA,
            pltpu.SemaphoreType.DMA,
            pltpu.SemaphoreType.DMA,
            pltpu.SemaphoreType.DMA,
        ],
    )


def _combine_body(y_hbm, destT_hbm, s_hbm, out_hbm,
                  buf_a, buf_b, ia_v, ib_v, sa_v, sem_i, sem_a, sem_b, sem_o):
    wid = lax.axis_index("c") * 16 + lax.axis_index("s")
    base = wid * TPW
    cps_i = [
        pltpu.async_copy(
            destT_hbm.at[k, pl.ds(base + c * CH, CH)],
            (ia_v, ib_v)[k].at[c], sem_i)
        for k in range(K) for c in range(NCH)
    ]
    cp_s = pltpu.async_copy(s_hbm.at[pl.ds(base, TPW)], sa_v, sem_i)
    for cp in cps_i:
        cp.wait()
    cps_a = []
    cps_b = []
    for c in range(NCH):
        sl = pl.ds(c * CH, CH)
        cps_a.append(pltpu.async_copy(y_hbm.at[ia_v.at[c]], buf_a.at[sl], sem_a))
        cps_b.append(pltpu.async_copy(y_hbm.at[ib_v.at[c]], buf_b.at[sl], sem_b))
    cp_s.wait()

    cps_o = []
    for c in range(NCH):
        cps_a[c].wait()
        cps_b[c].wait()

        def body(t, carry):
            sa = sa_v[t]
            for d in range(D // L):
                sl = pl.ds(d * L, L)
                b = buf_b[t, sl]
                buf_a[t, sl] = b + sa * (buf_a[t, sl] - b)
            return carry

        lax.fori_loop(c * CH, (c + 1) * CH, body, 0)
        sl = pl.ds(c * CH, CH)
        cps_o.append(pltpu.async_copy(
            buf_a.at[sl], out_hbm.at[pl.ds(base + c * CH, CH)], sem_o))
    for cp in cps_o:
        cp.wait()


def kernel(moe_inp, Wg, bg, W1, b1, W2, b2):
    s0, destT, meta = _gate_route(moe_inp, Wg, bg)
    xs = _build_dispatch()(moe_inp, destT)
    y_s = _grouped_ffn(meta, xs, W1, b1, W2, b2)
    return _build_combine()(y_s, destT, s0)


# speedup vs baseline: 1.0838x; 1.0838x over previous
import functools

import jax
import jax.numpy as jnp
from jax import lax
from jax.experimental import pallas as pl
from jax.experimental.pallas import tpu as pltpu
from jax.experimental.pallas import tpu_sc as plsc

E = 8
D = 768
F = 3072
K = 2
T = 2048
N = T * K
B = 256
NB = N // B + E
NP = NB * B
NEG = -1e30
NW = 32
TPW = T // NW
L = 16
CH = 32
NCH = TPW // CH
GS = 128
G = N // GS

FB = float(B)


def _gate_route_kernel(x_ref, wg_ref, bg_ref, s0_ref, destT_ref, meta_ref):
    x = x_ref[...]
    logits = jnp.dot(x, wg_ref[...], preferred_element_type=jnp.float32)
    logits = logits + bg_ref[...]
    l8i = (lax.broadcasted_iota(jnp.int32, (E, E), 0)
           <= lax.broadcasted_iota(jnp.int32, (E, E), 1)).astype(jnp.float32)
    m1 = jnp.max(logits, axis=1, keepdims=True)
    eq1 = (logits == m1).astype(jnp.float32)
    cs1 = jnp.dot(eq1, l8i, preferred_element_type=jnp.float32)
    oh_a = jnp.where(cs1 == 1.0, eq1, 0.0)
    l2 = logits + NEG * oh_a
    m2 = jnp.max(l2, axis=1, keepdims=True)
    eq2 = (l2 == m2).astype(jnp.float32)
    cs2 = jnp.dot(eq2, l8i, preferred_element_type=jnp.float32)
    oh_b = jnp.where(cs2 == 1.0, eq2, 0.0)
    s0 = 1.0 / (1.0 + jnp.exp(m2 - m1))
    s0_ref[...] = jnp.broadcast_to(s0, (T, L))
    oh = jnp.concatenate([oh_a, oh_b], axis=0)

    tril = (lax.broadcasted_iota(jnp.int32, (GS, GS), 0)
            >= lax.broadcasted_iota(jnp.int32, (GS, GS), 1)).astype(jnp.float32)
    segs = []
    csums = []
    totals = []
    for g in range(G):
        seg = oh[g * GS:(g + 1) * GS]
        cs = jnp.dot(tril, seg, preferred_element_type=jnp.float32)
        segs.append(seg)
        csums.append(cs)
        totals.append(cs[GS - 1:GS, :])
    tot = jnp.concatenate(totals, axis=0)
    trilg = (lax.broadcasted_iota(jnp.int32, (G, G), 0)
             > lax.broadcasted_iota(jnp.int32, (G, G), 1)).astype(jnp.float32)
    goff = jnp.dot(trilg, tot, preferred_element_type=jnp.float32)
    counts = jnp.sum(tot, axis=0, keepdims=True)
    nblk = jnp.floor((counts + (FB - 1.0)) / FB)
    padded = nblk * FB
    u8 = (lax.broadcasted_iota(jnp.int32, (E, E), 0)
          < lax.broadcasted_iota(jnp.int32, (E, E), 1)).astype(jnp.float32)
    poff = jnp.dot(padded, u8, preferred_element_type=jnp.float32)

    dparts = []
    for g in range(G):
        rank = csums[g] - segs[g] + goff[g:g + 1, :]
        dval = jnp.sum(segs[g] * (rank + poff), axis=1, keepdims=True)
        dparts.append(dval)
    dest = jnp.concatenate(dparts, axis=0)
    dm = jnp.concatenate(
        [dest[:T], dest[T:], jnp.zeros((T, 126), jnp.float32)], axis=1)
    dt = jnp.swapaxes(dm, 0, 1)
    destT_ref[...] = dt[:K, :].astype(jnp.int32)

    l8 = (lax.broadcasted_iota(jnp.int32, (E, E), 0)
          <= lax.broadcasted_iota(jnp.int32, (E, E), 1)).astype(jnp.float32)
    pbe = jnp.dot(nblk, l8, preferred_element_type=jnp.float32)
    nba = jnp.sum(nblk, axis=1, keepdims=True)
    jcol = lax.broadcasted_iota(jnp.int32, (NB, 1), 0).astype(jnp.float32)
    be_raw = jnp.sum((pbe <= jcol).astype(jnp.float32), axis=1, keepdims=True)
    be = jnp.minimum(be_raw, float(E - 1))
    last_e = jnp.sum(jnp.where(jcol == nba - 1.0, be, 0.0), axis=0, keepdims=True)
    bev = jnp.where(jcol < nba, be, last_e)
    mm = jnp.concatenate([bev, nba], axis=0)
    meta_ref[...] = jnp.broadcast_to(mm, (NB + 1, 128)).astype(jnp.int32)


def _gate_route(moe_inp, Wg, bg):
    return pl.pallas_call(
        _gate_route_kernel,
        grid=(1,),
        in_specs=[
            pl.BlockSpec((T, D), lambda i: (0, 0)),
            pl.BlockSpec((D, E), lambda i: (0, 0)),
            pl.BlockSpec((1, E), lambda i: (0, 0)),
        ],
        out_specs=[
            pl.BlockSpec((T, L), lambda i: (0, 0)),
            pl.BlockSpec((K, T), lambda i: (0, 0)),
            pl.BlockSpec((NB + 1, 128), lambda i: (0, 0)),
        ],
        out_shape=[
            jax.ShapeDtypeStruct((T, L), jnp.float32),
            jax.ShapeDtypeStruct((K, T), jnp.int32),
            jax.ShapeDtypeStruct((NB + 1, 128), jnp.int32),
        ],
    )(moe_inp, Wg, bg.reshape(1, E))


@functools.cache
def _build_dispatch():
    mesh = plsc.VectorSubcoreMesh(core_axis_name="c", subcore_axis_name="s")
    return pl.kernel(
        _dispatch_body,
        out_type=jax.ShapeDtypeStruct((NP, D), jnp.float32),
        mesh=mesh,
        scratch_types=[
            pltpu.VMEM((TPW, D), jnp.float32),
            pltpu.VMEM((NCH, CH), jnp.int32),
            pltpu.VMEM((NCH, CH), jnp.int32),
            pltpu.SemaphoreType.DMA,
            pltpu.SemaphoreType.DMA,
            pltpu.SemaphoreType.DMA,
            pltpu.SemaphoreType.DMA,
        ],
    )


def _dispatch_body(x_hbm, destT_hbm, xs_hbm, rows_v, ia_v, ib_v,
                   sem_x, sem_i, sem_a, sem_b):
    wid = lax.axis_index("c") * 16 + lax.axis_index("s")
    base = wid * TPW
    cps_x = [
        pltpu.async_copy(
            x_hbm.at[pl.ds(base + c * CH, CH)],
            rows_v.at[pl.ds(c * CH, CH)], sem_x)
        for c in range(NCH)
    ]
    cps_i = [
        pltpu.async_copy(
            destT_hbm.at[k, pl.ds(base + c * CH, CH)],
            (ia_v, ib_v)[k].at[c], sem_i)
        for k in range(K) for c in range(NCH)
    ]
    for cp in cps_i:
        cp.wait()
    outs = []
    for c in range(NCH):
        cps_x[c].wait()
        sl = pl.ds(c * CH, CH)
        outs.append(pltpu.async_copy(
            rows_v.at[sl], xs_hbm.at[ia_v.at[c]], sem_a))
        outs.append(pltpu.async_copy(
            rows_v.at[sl], xs_hbm.at[ib_v.at[c]], sem_b))
    for cp in outs:
        cp.wait()


def _ffn_kernel(meta_ref, x_ref, w1_ref, b1_ref, w2_ref, b2_ref, y_ref):
    j = pl.program_id(0)

    @pl.when(j < meta_ref[NB, 0])
    def _():
        x = x_ref[...]
        h = jnp.dot(x, w1_ref[0], preferred_element_type=jnp.float32)
        h = jnp.maximum(h + b1_ref[0], 0.0)
        y = jnp.dot(h, w2_ref[0], preferred_element_type=jnp.float32)
        y_ref[...] = y + b2_ref[0]


def _grouped_ffn(meta, xs, W1, b1, W2, b2):
    grid_spec = pltpu.PrefetchScalarGridSpec(
        num_scalar_prefetch=1,
        grid=(NB,),
        in_specs=[
            pl.BlockSpec((B, D), lambda j, m: (jnp.minimum(j, m[NB, 0] - 1), 0)),
            pl.BlockSpec((1, D, F), lambda j, m: (m[j, 0], 0, 0)),
            pl.BlockSpec((1, 1, F), lambda j, m: (m[j, 0], 0, 0)),
            pl.BlockSpec((1, F, D), lambda j, m: (m[j, 0], 0, 0)),
            pl.BlockSpec((1, 1, D), lambda j, m: (m[j, 0], 0, 0)),
        ],
        out_specs=pl.BlockSpec((B, D), lambda j, m: (jnp.minimum(j, m[NB, 0] - 1), 0)),
    )
    return pl.pallas_call(
        _ffn_kernel,
        grid_spec=grid_spec,
        out_shape=jax.ShapeDtypeStruct((NP, D), jnp.float32),
    )(meta, xs, W1, b1.reshape(E, 1, F), W2, b2.reshape(E, 1, D))


@functools.cache
def _build_combine():
    mesh = plsc.VectorSubcoreMesh(core_axis_name="c", subcore_axis_name="s")
    return pl.kernel(
        _combine_body,
        out_type=jax.ShapeDtypeStruct((T, D), jnp.float32),
        mesh=mesh,
        scratch_types=[
            pltpu.VMEM((TPW, D), jnp.float32),
            pltpu.VMEM((TPW, D), jnp.float32),
            pltpu.VMEM((NCH, CH), jnp.int32),
            pltpu.VMEM((NCH, CH), jnp.int32),
            pltpu.VMEM((TPW, L), jnp.float32),
            pltpu.SemaphoreType.DMA,
            pltpu.SemaphoreType.DMA,
            pltpu.SemaphoreType.DMA,
            pltpu.SemaphoreType.DMA,
        ],
    )


def _combine_body(y_hbm, destT_hbm, s_hbm, out_hbm,
                  buf_a, buf_b, ia_v, ib_v, sa_v, sem_i, sem_a, sem_b, sem_o):
    wid = lax.axis_index("c") * 16 + lax.axis_index("s")
    base = wid * TPW
    cps_i = [
        pltpu.async_copy(
            destT_hbm.at[k, pl.ds(base + c * CH, CH)],
            (ia_v, ib_v)[k].at[c], sem_i)
        for k in range(K) for c in range(NCH)
    ]
    cp_s = pltpu.async_copy(s_hbm.at[pl.ds(base, TPW)], sa_v, sem_i)
    for cp in cps_i:
        cp.wait()
    cps_a = []
    cps_b = []
    for c in range(NCH):
        sl = pl.ds(c * CH, CH)
        cps_a.append(pltpu.async_copy(y_hbm.at[ia_v.at[c]], buf_a.at[sl], sem_a))
        cps_b.append(pltpu.async_copy(y_hbm.at[ib_v.at[c]], buf_b.at[sl], sem_b))
    cp_s.wait()

    cps_o = []
    for c in range(NCH):
        cps_a[c].wait()
        cps_b[c].wait()

        def body(t, carry):
            sa = sa_v[t]
            for d in range(D // L):
                sl = pl.ds(d * L, L)
                b = buf_b[t, sl]
                buf_a[t, sl] = b + sa * (buf_a[t, sl] - b)
            return carry

        lax.fori_loop(c * CH, (c + 1) * CH, body, 0)
        sl = pl.ds(c * CH, CH)
        cps_o.append(pltpu.async_copy(
            buf_a.at[sl], out_hbm.at[pl.ds(base + c * CH, CH)], sem_o))
    for cp in cps_o:
        cp.wait()


def kernel(moe_inp, Wg, bg, W1, b1, W2, b2):
    s0, destT, meta = _gate_route(moe_inp, Wg, bg)
    xs = _build_dispatch()(moe_inp, destT)
    y_s = _grouped_ffn(meta, xs, W1, b1, W2, b2)
    return _build_combine()(y_s, destT, s0)


# speedup vs baseline: 1.0866x; 1.0026x over previous
import functools

import jax
import jax.numpy as jnp
from jax import lax
from jax.experimental import pallas as pl
from jax.experimental.pallas import tpu as pltpu
from jax.experimental.pallas import tpu_sc as plsc

E = 8
D = 768
F = 3072
K = 2
T = 2048
N = T * K
B = 256
NB = N // B + E
NP = NB * B
NEG = -1e30
NW = 32
TPW = T // NW
L = 16
CH = 32
NCH = TPW // CH
GS = 128
G = N // GS

FB = float(B)


def _gate_route_kernel(x_ref, wg_ref, bg_ref, s0_ref, destT_ref, meta_ref,
                       oh_s):
    i = pl.program_id(0)
    x = x_ref[...]
    logits = jnp.dot(x, wg_ref[...], preferred_element_type=jnp.float32)
    logits = logits + bg_ref[...]
    l8i = (lax.broadcasted_iota(jnp.int32, (E, E), 0)
           <= lax.broadcasted_iota(jnp.int32, (E, E), 1)).astype(jnp.float32)
    m1 = jnp.max(logits, axis=1, keepdims=True)
    eq1 = (logits == m1).astype(jnp.float32)
    cs1 = jnp.dot(eq1, l8i, preferred_element_type=jnp.float32)
    oh_a = jnp.where(cs1 == 1.0, eq1, 0.0)
    l2 = logits + NEG * oh_a
    m2 = jnp.max(l2, axis=1, keepdims=True)
    eq2 = (l2 == m2).astype(jnp.float32)
    cs2 = jnp.dot(eq2, l8i, preferred_element_type=jnp.float32)
    oh_b = jnp.where(cs2 == 1.0, eq2, 0.0)
    s0 = 1.0 / (1.0 + jnp.exp(m2 - m1))
    s0_ref[...] = jnp.broadcast_to(s0, (T // 2, L))
    half = pl.ds(i * (T // 2), T // 2)
    oh_s[half] = oh_a
    oh_s[pl.ds(T + i * (T // 2), T // 2)] = oh_b

    @pl.when(i == 1)
    def _finalize():
        _route_tail(oh_s, destT_ref, meta_ref)


def _route_tail(oh_s, destT_ref, meta_ref):
    oh = oh_s[...]

    tril = (lax.broadcasted_iota(jnp.int32, (GS, GS), 0)
            >= lax.broadcasted_iota(jnp.int32, (GS, GS), 1)).astype(jnp.float32)
    segs = []
    csums = []
    totals = []
    for g in range(G):
        seg = oh[g * GS:(g + 1) * GS]
        cs = jnp.dot(tril, seg, preferred_element_type=jnp.float32)
        segs.append(seg)
        csums.append(cs)
        totals.append(cs[GS - 1:GS, :])
    tot = jnp.concatenate(totals, axis=0)
    trilg = (lax.broadcasted_iota(jnp.int32, (G, G), 0)
             > lax.broadcasted_iota(jnp.int32, (G, G), 1)).astype(jnp.float32)
    goff = jnp.dot(trilg, tot, preferred_element_type=jnp.float32)
    counts = jnp.sum(tot, axis=0, keepdims=True)
    nblk = jnp.floor((counts + (FB - 1.0)) / FB)
    padded = nblk * FB
    u8 = (lax.broadcasted_iota(jnp.int32, (E, E), 0)
          < lax.broadcasted_iota(jnp.int32, (E, E), 1)).astype(jnp.float32)
    poff = jnp.dot(padded, u8, preferred_element_type=jnp.float32)

    dparts = []
    for g in range(G):
        rank = csums[g] - segs[g] + goff[g:g + 1, :]
        dval = jnp.sum(segs[g] * (rank + poff), axis=1, keepdims=True)
        dparts.append(dval)
    dest = jnp.concatenate(dparts, axis=0)
    dm = jnp.concatenate(
        [dest[:T], dest[T:], jnp.zeros((T, 126), jnp.float32)], axis=1)
    dt = jnp.swapaxes(dm, 0, 1)
    destT_ref[...] = dt[:K, :].astype(jnp.int32)

    l8 = (lax.broadcasted_iota(jnp.int32, (E, E), 0)
          <= lax.broadcasted_iota(jnp.int32, (E, E), 1)).astype(jnp.float32)
    pbe = jnp.dot(nblk, l8, preferred_element_type=jnp.float32)
    nba = jnp.sum(nblk, axis=1, keepdims=True)
    jcol = lax.broadcasted_iota(jnp.int32, (NB, 1), 0).astype(jnp.float32)
    be_raw = jnp.sum((pbe <= jcol).astype(jnp.float32), axis=1, keepdims=True)
    be = jnp.minimum(be_raw, float(E - 1))
    last_e = jnp.sum(jnp.where(jcol == nba - 1.0, be, 0.0), axis=0, keepdims=True)
    bev = jnp.where(jcol < nba, be, last_e)
    mm = jnp.concatenate([bev, nba], axis=0)
    meta_ref[...] = jnp.broadcast_to(mm, (NB + 1, 128)).astype(jnp.int32)


def _gate_route(moe_inp, Wg, bg):
    return pl.pallas_call(
        _gate_route_kernel,
        grid=(2,),
        in_specs=[
            pl.BlockSpec((T // 2, D), lambda i: (i, 0)),
            pl.BlockSpec((D, E), lambda i: (0, 0)),
            pl.BlockSpec((1, E), lambda i: (0, 0)),
        ],
        out_specs=[
            pl.BlockSpec((T // 2, L), lambda i: (i, 0)),
            pl.BlockSpec((K, T), lambda i: (0, 0)),
            pl.BlockSpec((NB + 1, 128), lambda i: (0, 0)),
        ],
        out_shape=[
            jax.ShapeDtypeStruct((T, L), jnp.float32),
            jax.ShapeDtypeStruct((K, T), jnp.int32),
            jax.ShapeDtypeStruct((NB + 1, 128), jnp.int32),
        ],
        scratch_shapes=[pltpu.VMEM((N, E), jnp.float32)],
    )(moe_inp, Wg, bg.reshape(1, E))


@functools.cache
def _build_dispatch():
    mesh = plsc.VectorSubcoreMesh(core_axis_name="c", subcore_axis_name="s")
    return pl.kernel(
        _dispatch_body,
        out_type=jax.ShapeDtypeStruct((NP, D), jnp.float32),
        mesh=mesh,
        scratch_types=[
            pltpu.VMEM((TPW, D), jnp.float32),
            pltpu.VMEM((NCH, CH), jnp.int32),
            pltpu.VMEM((NCH, CH), jnp.int32),
            pltpu.SemaphoreType.DMA,
            pltpu.SemaphoreType.DMA,
            pltpu.SemaphoreType.DMA,
            pltpu.SemaphoreType.DMA,
        ],
    )


def _dispatch_body(x_hbm, destT_hbm, xs_hbm, rows_v, ia_v, ib_v,
                   sem_x, sem_i, sem_a, sem_b):
    wid = lax.axis_index("c") * 16 + lax.axis_index("s")
    base = wid * TPW
    cps_x = [
        pltpu.async_copy(
            x_hbm.at[pl.ds(base + c * CH, CH)],
            rows_v.at[pl.ds(c * CH, CH)], sem_x)
        for c in range(NCH)
    ]
    cps_i = [
        pltpu.async_copy(
            destT_hbm.at[k, pl.ds(base + c * CH, CH)],
            (ia_v, ib_v)[k].at[c], sem_i)
        for k in range(K) for c in range(NCH)
    ]
    for cp in cps_i:
        cp.wait()
    outs = []
    for c in range(NCH):
        cps_x[c].wait()
        sl = pl.ds(c * CH, CH)
        outs.append(pltpu.async_copy(
            rows_v.at[sl], xs_hbm.at[ia_v.at[c]], sem_a))
        outs.append(pltpu.async_copy(
            rows_v.at[sl], xs_hbm.at[ib_v.at[c]], sem_b))
    for cp in outs:
        cp.wait()


def _ffn_kernel(meta_ref, x_ref, w1_ref, b1_ref, w2_ref, b2_ref, y_ref):
    j = pl.program_id(0)

    @pl.when(j < meta_ref[NB, 0])
    def _():
        x = x_ref[...]
        h = jnp.dot(x, w1_ref[0], preferred_element_type=jnp.float32)
        h = jnp.maximum(h + b1_ref[0], 0.0)
        y = jnp.dot(h, w2_ref[0], preferred_element_type=jnp.float32)
        y_ref[...] = y + b2_ref[0]


def _grouped_ffn(meta, xs, W1, b1, W2, b2):
    grid_spec = pltpu.PrefetchScalarGridSpec(
        num_scalar_prefetch=1,
        grid=(NB,),
        in_specs=[
            pl.BlockSpec((B, D), lambda j, m: (jnp.minimum(j, m[NB, 0] - 1), 0)),
            pl.BlockSpec((1, D, F), lambda j, m: (m[j, 0], 0, 0)),
            pl.BlockSpec((1, 1, F), lambda j, m: (m[j, 0], 0, 0)),
            pl.BlockSpec((1, F, D), lambda j, m: (m[j, 0], 0, 0)),
            pl.BlockSpec((1, 1, D), lambda j, m: (m[j, 0], 0, 0)),
        ],
        out_specs=pl.BlockSpec((B, D), lambda j, m: (jnp.minimum(j, m[NB, 0] - 1), 0)),
    )
    return pl.pallas_call(
        _ffn_kernel,
        grid_spec=grid_spec,
        out_shape=jax.ShapeDtypeStruct((NP, D), jnp.float32),
    )(meta, xs, W1, b1.reshape(E, 1, F), W2, b2.reshape(E, 1, D))


@functools.cache
def _build_combine():
    mesh = plsc.VectorSubcoreMesh(core_axis_name="c", subcore_axis_name="s")
    return pl.kernel(
        _combine_body,
        out_type=jax.ShapeDtypeStruct((T, D), jnp.float32),
        mesh=mesh,
        scratch_types=[
            pltpu.VMEM((TPW, D), jnp.float32),
            pltpu.VMEM((TPW, D), jnp.float32),
            pltpu.VMEM((NCH, CH), jnp.int32),
            pltpu.VMEM((NCH, CH), jnp.int32),
            pltpu.VMEM((TPW, L), jnp.float32),
            pltpu.SemaphoreType.DMA,
            pltpu.SemaphoreType.DMA,
            pltpu.SemaphoreType.DMA,
            pltpu.SemaphoreType.DMA,
        ],
    )


def _combine_body(y_hbm, destT_hbm, s_hbm, out_hbm,
                  buf_a, buf_b, ia_v, ib_v, sa_v, sem_i, sem_a, sem_b, sem_o):
    wid = lax.axis_index("c") * 16 + lax.axis_index("s")
    base = wid * TPW
    cps_i = [
        pltpu.async_copy(
            destT_hbm.at[k, pl.ds(base + c * CH, CH)],
            (ia_v, ib_v)[k].at[c], sem_i)
        for k in range(K) for c in range(NCH)
    ]
    cp_s = pltpu.async_copy(s_hbm.at[pl.ds(base, TPW)], sa_v, sem_i)
    for cp in cps_i:
        cp.wait()
    cps_a = []
    cps_b = []
    for c in range(NCH):
        sl = pl.ds(c * CH, CH)
        cps_a.append(pltpu.async_copy(y_hbm.at[ia_v.at[c]], buf_a.at[sl], sem_a))
        cps_b.append(pltpu.async_copy(y_hbm.at[ib_v.at[c]], buf_b.at[sl], sem_b))
    cp_s.wait()

    cps_o = []
    for c in range(NCH):
        cps_a[c].wait()
        cps_b[c].wait()

        def body(t, carry):
            sa = sa_v[t]
            for d in range(D // L):
                sl = pl.ds(d * L, L)
                b = buf_b[t, sl]
                buf_a[t, sl] = b + sa * (buf_a[t, sl] - b)
            return carry

        lax.fori_loop(c * CH, (c + 1) * CH, body, 0)
        sl = pl.ds(c * CH, CH)
        cps_o.append(pltpu.async_copy(
            buf_a.at[sl], out_hbm.at[pl.ds(base + c * CH, CH)], sem_o))
    for cp in cps_o:
        cp.wait()


def kernel(moe_inp, Wg, bg, W1, b1, W2, b2):
    s0, destT, meta = _gate_route(moe_inp, Wg, bg)
    xs = _build_dispatch()(moe_inp, destT)
    y_s = _grouped_ffn(meta, xs, W1, b1, W2, b2)
    return _build_combine()(y_s, destT, s0)
